# baseline (speedup 1.0000x reference)
import functools

import jax
import jax.numpy as jnp
from jax import lax
from jax.experimental import pallas as pl
from jax.experimental.pallas import tpu as pltpu
from jax.experimental.pallas import tpu_sc as plsc

_VOCAB = 1000000
_EMBED_DIM = 64
_BATCH = 16384
_HIST = 50
_HPAD = 56
_EPAD = 128

_info = plsc.get_sparse_core_info()
_NC = _info.num_cores
_NS = _info.num_subcores
_NW = _NC * _NS
_BPW = _BATCH // _NW
_CB = 2
_NCHUNK = _BPW // _CB
_CR = _CB * _HIST
_NBUF = 4
assert _NCHUNK % _NBUF == 0

_mesh = plsc.VectorSubcoreMesh(core_axis_name="c", subcore_axis_name="s")

_RCH = 168
_DCH = 186
_DROWS = _RCH * _DCH
_DTAIL = _VOCAB - _NW * _DROWS
_DNB = 3
assert _DCH % _DNB == 0 and _DROWS % 8 == 0 and _DTAIL % 8 == 0


@functools.partial(
    pl.kernel,
    mesh=_mesh,
    out_type=jax.ShapeDtypeStruct((_VOCAB, 2 * _EMBED_DIM), jnp.float32),
    scratch_types=[
        [pltpu.VMEM((_RCH, _EMBED_DIM), jnp.float32)] * _DNB,
        [pltpu.VMEM((_RCH, 2 * _EMBED_DIM), jnp.float32)] * _DNB,
        [pltpu.SemaphoreType.DMA] * _DNB,
        [pltpu.SemaphoreType.DMA] * _DNB,
    ],
    compiler_params=pltpu.CompilerParams(use_tc_tiling_on_sc=True),
)
def _repack_kernel(table_hbm, out_hbm, ba, bb, rsems, wsems):
    wid = lax.axis_index("s") * _NC + lax.axis_index("c")
    r0w = wid * _DROWS

    def r_cp(c, b):
        return pltpu.make_async_copy(
            table_hbm.at[pl.ds(r0w + c * _RCH, _RCH)], ba[b], rsems[b])

    def w_cp(c, b):
        return pltpu.make_async_copy(
            bb[b], out_hbm.at[pl.ds(r0w + c * _RCH, _RCH)], wsems[b])

    def repack(b, rows=_RCH):
        def rp(i, _):
            for k in range(_EMBED_DIM // 16):
                bb[b][i, pl.ds(16 * k, 16)] = ba[b][i, pl.ds(16 * k, 16)]
            return 0
        lax.fori_loop(0, rows, rp, 0)

    for b in range(_DNB - 1):
        r_cp(b, b).start()

    def body(g, _):
        for b in range(_DNB):
            c = g * _DNB + b
            r_cp(c, b).wait()

            @pl.when(c >= _DNB)
            def _(c=c, b=b):
                w_cp(c - _DNB, b).wait()
            repack(b)
            w_cp(c, b).start()
            n = c + _DNB - 1
            nb = (b - 1) % _DNB

            @pl.when(n < _DCH)
            def _(n=n, nb=nb):
                r_cp(n, nb).start()
        return 0

    lax.fori_loop(0, _DCH // _DNB, body, 0)
    for b in range(_DNB):
        w_cp(_DCH - _DNB + b, b).wait()

    @pl.when(wid == _NW - 1)
    def _():
        t0 = _NW * _DROWS
        pltpu.sync_copy(table_hbm.at[pl.ds(t0, _DTAIL)],
                        ba[0].at[pl.ds(0, _DTAIL)])
        repack(0, _DTAIL)
        pltpu.sync_copy(bb[0].at[pl.ds(0, _DTAIL)],
                        out_hbm.at[pl.ds(t0, _DTAIL)])


@functools.partial(
    pl.kernel,
    mesh=_mesh,
    out_type=jax.ShapeDtypeStruct((_BATCH, _HPAD, _EPAD), jnp.float32),
    scratch_types=[
        pltpu.VMEM((_NCHUNK, _CR), jnp.int32),
        [pltpu.VMEM((_CR, _EPAD), jnp.float32)] * _NBUF,
        [pltpu.SemaphoreType.DMA] * _NBUF,
        [pltpu.SemaphoreType.DMA] * _NBUF,
    ],
    compiler_params=pltpu.CompilerParams(use_tc_tiling_on_sc=False),
)
def _gather_kernel(idx_hbm, table_hbm, out_hbm, idx_v, rows, gsems, wsems):
    wid = lax.axis_index("s") * _NC + lax.axis_index("c")
    base = wid * _BPW

    pltpu.sync_copy(idx_hbm.at[pl.ds(wid * _NCHUNK, _NCHUNK)], idx_v)

    def g_start(c, b):
        pltpu.async_copy(table_hbm.at[idx_v.at[c]], rows[b], gsems[b])

    def g_wait(c, b):
        pltpu.make_async_copy(table_hbm.at[idx_v.at[c]], rows[b],
                              gsems[b]).wait()

    def _w_copies(c, b):
        for k in range(_CB):
            yield pltpu.make_async_copy(
                rows[b].at[pl.ds(k * _HIST, _HIST), pl.ds(0, _EMBED_DIM)],
                out_hbm.at[base + c * _CB + k,
                           pl.ds(0, _HIST), pl.ds(0, _EMBED_DIM)],
                wsems[b])

    def w_start(c, b):
        for cp in _w_copies(c, b):
            cp.start()

    def w_wait(c, b):
        for cp in _w_copies(c, b):
            cp.wait()

    for b in range(_NBUF - 1):
        g_start(b, b)

    def body(g, _):
        for b in range(_NBUF):
            c = g * _NBUF + b
            g_wait(c, b)
            w_start(c, b)
            n = c + _NBUF - 1
            nb = (b - 1) % _NBUF

            @pl.when(n < _NCHUNK)
            def _(c=c, n=n, nb=nb):
                @pl.when(n >= _NBUF)
                def _():
                    w_wait(n - _NBUF, nb)
                g_start(n, nb)
        return 0

    lax.fori_loop(0, _NCHUNK // _NBUF, body, 0)
    for b in range(_NBUF):
        c = _NCHUNK - _NBUF + b
        w_wait(c, b)


def kernel(input_ids, table):
    idx = input_ids.astype(jnp.int32).reshape(_BATCH * _HIST // _CR, _CR)
    table_pad = _repack_kernel(table)
    out = _gather_kernel(idx, table_pad)
    return out[:, :_HIST, :_EMBED_DIM]


# speedup vs baseline: 1.1870x; 1.1870x over previous
import functools

import jax
import jax.numpy as jnp
from jax import lax
from jax.experimental import pallas as pl
from jax.experimental.pallas import tpu as pltpu
from jax.experimental.pallas import tpu_sc as plsc

_VOCAB = 1000000
_EMBED_DIM = 64
_BATCH = 16384
_HIST = 50
_HPAD = 56
_EPAD = 128

_info = plsc.get_sparse_core_info()
_NC = _info.num_cores
_NS = _info.num_subcores
_NW = _NC * _NS
_BPW = _BATCH // _NW
_CB = 4
_NCHUNK = _BPW // _CB
_CR = _CB * _HIST
_NBUF = 8
assert _NCHUNK % _NBUF == 0

_mesh = plsc.VectorSubcoreMesh(core_axis_name="c", subcore_axis_name="s")

@functools.partial(
    pl.kernel,
    mesh=_mesh,
    out_type=jax.ShapeDtypeStruct((_BATCH, _HPAD, _EPAD), jnp.float32),
    scratch_types=[
        pltpu.VMEM((_NCHUNK, _CR), jnp.int32),
        [pltpu.VMEM((_CR, _EMBED_DIM), jnp.float32)] * _NBUF,
        [pltpu.SemaphoreType.DMA] * _NBUF,
        [pltpu.SemaphoreType.DMA] * _NBUF,
    ],
    compiler_params=pltpu.CompilerParams(use_tc_tiling_on_sc=False),
)
def _gather_kernel(idx_hbm, table_hbm, out_hbm, idx_v, rows, gsems, wsems):
    wid = lax.axis_index("s") * _NC + lax.axis_index("c")
    base = wid * _BPW

    pltpu.sync_copy(idx_hbm.at[pl.ds(wid * _NCHUNK, _NCHUNK)], idx_v)

    def g_start(c, b):
        pltpu.async_copy(table_hbm.at[idx_v.at[c]], rows[b], gsems[b])

    def g_wait(c, b):
        pltpu.make_async_copy(table_hbm.at[idx_v.at[c]], rows[b],
                              gsems[b]).wait()

    def _w_copies(c, b):
        for k in range(_CB):
            yield pltpu.make_async_copy(
                rows[b].at[pl.ds(k * _HIST, _HIST)],
                out_hbm.at[base + c * _CB + k,
                           pl.ds(0, _HIST), pl.ds(0, _EMBED_DIM)],
                wsems[b])

    def w_start(c, b):
        for cp in _w_copies(c, b):
            cp.start()

    def w_wait(c, b):
        for cp in _w_copies(c, b):
            cp.wait()

    for b in range(_NBUF - 1):
        g_start(b, b)

    def body(g, _):
        for b in range(_NBUF):
            c = g * _NBUF + b
            g_wait(c, b)
            w_start(c, b)
            n = c + _NBUF - 1
            nb = (b - 1) % _NBUF

            @pl.when(n < _NCHUNK)
            def _(c=c, n=n, nb=nb):
                @pl.when(n >= _NBUF)
                def _():
                    w_wait(n - _NBUF, nb)
                g_start(n, nb)
        return 0

    lax.fori_loop(0, _NCHUNK // _NBUF, body, 0)
    for b in range(_NBUF):
        c = _NCHUNK - _NBUF + b
        w_wait(c, b)


def kernel(input_ids, table):
    idx = input_ids.astype(jnp.int32).reshape(_BATCH * _HIST // _CR, _CR)
    out = _gather_kernel(idx, table)
    return out[:, :_HIST, :_EMBED_DIM]
